# baseline (speedup 1.0000x reference)
import functools
import math

import jax
import jax.numpy as jnp
from jax import lax
from jax.experimental import pallas as pl
from jax.experimental.pallas import tpu as pltpu
from jax.experimental.pallas import tpu_sc as plsc

D_MODEL = 64
SCALE = math.sqrt(D_MODEL)
NC = 2
NS = 16
NW = NC * NS
B = 4096 * 200
BPW = B // NW
CHUNK = 1024
NCHUNK = BPW // CHUNK
SUB = 128
NSUB = CHUNK // SUB

_mesh = plsc.VectorSubcoreMesh(core_axis_name="c", subcore_axis_name="s")


@functools.partial(
    pl.kernel,
    out_type=jax.ShapeDtypeStruct((B, D_MODEL), jnp.float32),
    mesh=_mesh,
    scratch_types=[
        pltpu.VMEM((CHUNK,), jnp.int32),
        pltpu.VMEM((CHUNK, D_MODEL), jnp.float32),
        pltpu.SemaphoreType.DMA,
    ],
    compiler_params=pltpu.CompilerParams(use_tc_tiling_on_sc=False),
)
def _emb_lookup(x_hbm, lut_hbm, out_hbm, idx_v, rows_v, sem):
    wid = lax.axis_index("s") * NC + lax.axis_index("c")
    base = wid * BPW

    def chunk_body(c, carry):
        off = base + c * CHUNK
        pltpu.sync_copy(x_hbm.at[pl.ds(off, CHUNK)], idx_v)
        copies = []
        for j in range(NSUB):
            copies.append(
                pltpu.async_copy(
                    lut_hbm.at[idx_v.at[pl.ds(j * SUB, SUB)]],
                    rows_v.at[pl.ds(j * SUB, SUB)],
                    sem,
                )
            )
        for cop in copies:
            cop.wait()

        def row_body(i, rcarry):
            for j in range(D_MODEL // 16):
                rows_v[i, pl.ds(j * 16, 16)] = rows_v[i, pl.ds(j * 16, 16)] * SCALE
            return rcarry

        lax.fori_loop(0, CHUNK, row_body, 0)
        pltpu.sync_copy(rows_v, out_hbm.at[pl.ds(off, CHUNK)])
        return carry

    lax.fori_loop(0, NCHUNK, chunk_body, 0)


def kernel(x, lut):
    xf = x.reshape(-1).astype(jnp.int32)
    out = _emb_lookup(xf, lut)
    return out.reshape(x.shape + (D_MODEL,))


# speedup vs baseline: 1.0940x; 1.0940x over previous
import functools
import math

import jax
import jax.numpy as jnp
from jax import lax
from jax.experimental import pallas as pl
from jax.experimental.pallas import tpu as pltpu
from jax.experimental.pallas import tpu_sc as plsc

D_MODEL = 64
SCALE = math.sqrt(D_MODEL)
NC = 2
NS = 16
NW = NC * NS
B = 4096 * 200
BPW = B // NW
CHUNK = 512
NCHUNK = BPW // CHUNK
SUB = 128
NSUB = CHUNK // SUB
LAST_BUF = (NCHUNK - 1) % 2

_mesh = plsc.VectorSubcoreMesh(core_axis_name="c", subcore_axis_name="s")


@functools.partial(
    pl.kernel,
    out_type=jax.ShapeDtypeStruct((B, D_MODEL), jnp.float32),
    mesh=_mesh,
    scratch_types=[
        pltpu.VMEM((2, CHUNK), jnp.int32),
        pltpu.VMEM((2, CHUNK, D_MODEL), jnp.float32),
        pltpu.SemaphoreType.DMA,
        pltpu.SemaphoreType.DMA,
    ],
    compiler_params=pltpu.CompilerParams(use_tc_tiling_on_sc=False),
)
def _emb_lookup(x_hbm, lut_hbm, out_hbm, idx_v, rows_v, gsem, wsem):
    wid = lax.axis_index("s") * NC + lax.axis_index("c")
    base = wid * BPW

    def fire(c, buf):
        off = base + c * CHUNK
        idx_b = idx_v.at[buf]
        rows_b = rows_v.at[buf]
        pltpu.sync_copy(x_hbm.at[pl.ds(off, CHUNK)], idx_b)
        for j in range(NSUB):
            pltpu.async_copy(
                lut_hbm.at[idx_b.at[pl.ds(j * SUB, SUB)]],
                rows_b.at[pl.ds(j * SUB, SUB)],
                gsem,
            )

    fire(0, 0)

    @pl.loop(0, NCHUNK, step=2)
    def _(cc):
        for b in range(2):
            c = cc + b
            cur = rows_v.at[b]
            nxt = rows_v.at[1 - b]

            @pl.when(c > 0)
            def _():
                pltpu.make_async_copy(
                    nxt, out_hbm.at[pl.ds(0, CHUNK)], wsem
                ).wait()

            @pl.when(c + 1 < NCHUNK)
            def _():
                fire(c + 1, 1 - b)

            pltpu.make_async_copy(lut_hbm.at[pl.ds(0, CHUNK)], cur, gsem).wait()

            @plsc.parallel_loop(0, CHUNK, unroll=4)
            def _(i):
                for j in range(D_MODEL // 16):
                    cur[i, pl.ds(j * 16, 16)] = cur[i, pl.ds(j * 16, 16)] * SCALE

            pltpu.async_copy(cur, out_hbm.at[pl.ds(base + c * CHUNK, CHUNK)], wsem)

    pltpu.make_async_copy(
        rows_v.at[LAST_BUF], out_hbm.at[pl.ds(0, CHUNK)], wsem
    ).wait()


def kernel(x, lut):
    xf = x.reshape(-1).astype(jnp.int32)
    out = _emb_lookup(xf, lut)
    return out.reshape(x.shape + (D_MODEL,))
